# baseline (speedup 1.0000x reference)
import functools

import jax
import jax.numpy as jnp
from jax import lax
from jax.experimental import pallas as pl
from jax.experimental.pallas import tpu as pltpu
from jax.experimental.pallas import tpu_sc as plsc

_NUM_OBJS = 1000
_NUM_RELS = 64
_BATCH = 16384
_L = 16
_IDX_CHUNK = 128


@functools.lru_cache(maxsize=None)
def _build(num_cores: int, num_subcores: int):
    nw = num_cores * num_subcores
    bpw = _BATCH // nw
    n_chunks = bpw // _IDX_CHUNK
    mesh = plsc.VectorSubcoreMesh(
        core_axis_name="c", subcore_axis_name="s",
        num_cores=num_cores, num_subcores=num_subcores)

    @functools.partial(
        pl.kernel,
        out_type=jax.ShapeDtypeStruct((nw, _NUM_RELS * bpw), jnp.float32),
        mesh=mesh,
        scratch_types=[
            pltpu.VMEM((bpw,), jnp.int32),
            pltpu.VMEM((bpw,), jnp.int32),
            pltpu.VMEM((_NUM_RELS * n_chunks, _IDX_CHUNK), jnp.int32),
            pltpu.VMEM((_NUM_RELS * bpw,), jnp.float32),
            pltpu.SemaphoreType.DMA,
        ],
    )
    def k(l0_hbm, l1_hbm, table_hbm, out_hbm, l0_v, l1_v, widx_v, buf_v,
          sem):
        wid = lax.axis_index("s") * num_cores + lax.axis_index("c")
        base = wid * bpw
        pltpu.sync_copy(l0_hbm.at[pl.ds(base, bpw)], l0_v)
        pltpu.sync_copy(l1_hbm.at[pl.ds(base, bpw)], l1_v)
        for j in range(bpw // _L):
            fused = (l0_v[pl.ds(j * _L, _L)] * _NUM_OBJS
                     + l1_v[pl.ds(j * _L, _L)])
            widx_v[(j * _L) // _IDX_CHUNK,
                   pl.ds((j * _L) % _IDX_CHUNK, _L)] = fused
        def rep(c, _):
            for q in range(n_chunks):
                for t in range(_IDX_CHUNK // _L):
                    widx_v[c * n_chunks + q, pl.ds(t * _L, _L)] = (
                        widx_v[q, pl.ds(t * _L, _L)] + c * (_NUM_OBJS ** 2))
            return 0
        lax.fori_loop(1, _NUM_RELS, rep, 0)
        def fire(c, _):
            for q in range(n_chunks):
                pltpu.async_copy(
                    table_hbm.at[widx_v.at[c * n_chunks + q]],
                    buf_v.at[pl.ds(c * bpw + q * _IDX_CHUNK, _IDX_CHUNK)],
                    sem)
            return 0
        lax.fori_loop(0, _NUM_RELS, fire, 0)
        pltpu.make_async_copy(
            table_hbm.at[pl.ds(0, _NUM_RELS * bpw)], buf_v, sem).wait()
        pltpu.sync_copy(buf_v, out_hbm.at[wid])

    return k


def kernel(labels, obj_baseline):
    info = plsc.get_sparse_core_info()
    nw = info.num_cores * info.num_subcores
    k = _build(info.num_cores, info.num_subcores)
    out3 = k(labels[:, 0], labels[:, 1], obj_baseline.T.reshape(-1))
    bpw = _BATCH // nw
    out3 = out3.reshape(nw, _NUM_RELS, bpw)
    return jnp.transpose(out3, (0, 2, 1)).reshape(_BATCH, _NUM_RELS)
